# baseline (speedup 1.0000x reference)
import functools

import jax
import jax.numpy as jnp
from jax import lax
from jax.experimental import pallas as pl
from jax.experimental.pallas import tpu as pltpu
from jax.experimental.pallas import tpu_sc as plsc

VOCAB = 49408
EMBED = 256
BATCH = 4096
SEQ = 77

NUM_CORES = 2
NUM_SUBCORES = 16
NUM_WORKERS = NUM_CORES * NUM_SUBCORES
BLOCK = BATCH // NUM_WORKERS
NBUF = 3
LANES = 16


def _body(idx_hbm, token_hbm, pos_hbm, out_hbm, idx_v, pos_v,
          b0, b1, b2, g0, g1, g2, s0, s1, s2):
    bufs = (b0, b1, b2)
    gsems = (g0, g1, g2)
    ssems = (s0, s1, s2)

    wid = lax.axis_index("s") * NUM_CORES + lax.axis_index("c")
    base = wid * BLOCK

    pltpu.sync_copy(idx_hbm.at[wid], idx_v)
    pltpu.sync_copy(pos_hbm, pos_v)

    def gather_start(s, b):
        pltpu.async_copy(token_hbm.at[idx_v.at[s]], bufs[b], gsems[b])

    def gather_wait(b):
        pltpu.make_async_copy(token_hbm.at[pl.ds(0, BLOCK)], bufs[b], gsems[b]).wait()

    def store_wait(b):
        pltpu.make_async_copy(bufs[b], out_hbm.at[0, pl.ds(base, BLOCK)], ssems[b]).wait()

    def add_pos_store(s, b):
        buf = bufs[b]
        regs = [pos_v[s, pl.ds(c * LANES, LANES)] for c in range(EMBED // LANES)]
        half = BLOCK // 2

        @plsc.parallel_loop(0, half, 1, unroll=4)
        def _(i):
            for c in range(EMBED // LANES):
                plsc.addupdate(buf.at[i, pl.ds(c * LANES, LANES)], regs[c])

        pltpu.async_copy(buf.at[pl.ds(0, half)],
                         out_hbm.at[s, pl.ds(base, half)], ssems[b])

        @plsc.parallel_loop(half, BLOCK, 1, unroll=4)
        def _(i):
            for c in range(EMBED // LANES):
                plsc.addupdate(buf.at[i, pl.ds(c * LANES, LANES)], regs[c])

        pltpu.async_copy(buf.at[pl.ds(half, half)],
                         out_hbm.at[s, pl.ds(base + half, half)], ssems[b])

    for b in range(NBUF - 1):
        gather_start(b, b)

    for b in range(NBUF):
        s = b
        if s == 0:
            gather_start(NBUF - 1, NBUF - 1)
        else:
            store_wait((b - 1) % NBUF)
            gather_start(s + NBUF - 1, (b - 1) % NBUF)
        gather_wait(b)
        add_pos_store(s, b)

    def group(g, carry):
        s0_ = g * NBUF
        for b in range(NBUF):
            s = s0_ + b
            store_wait((b - 1) % NBUF)
            gather_start(s + NBUF - 1, (b - 1) % NBUF)
            gather_wait(b)
            add_pos_store(s, b)
        return carry

    lax.fori_loop(1, (SEQ - (NBUF - 1) - NBUF) // NBUF + 1, group, 0)

    for s in range(SEQ - ((SEQ - NBUF) % NBUF), SEQ):
        b = s % NBUF
        store_wait((b - 1) % NBUF)
        gather_wait(b)
        add_pos_store(s, b)

    store_wait((SEQ - 1) % NBUF)


@jax.jit
def _run(idx_blocks, token_table, pos_table):
    mesh = plsc.VectorSubcoreMesh(core_axis_name="c", subcore_axis_name="s")
    f = functools.partial(
        pl.kernel,
        out_type=jax.ShapeDtypeStruct((SEQ, BATCH, EMBED), jnp.float32),
        mesh=mesh,
        scratch_types=[
            pltpu.VMEM((SEQ, BLOCK), jnp.int32),
            pltpu.VMEM((SEQ, EMBED), jnp.float32),
        ] + [pltpu.VMEM((BLOCK, EMBED), jnp.float32)] * NBUF
          + [pltpu.SemaphoreType.DMA] * (2 * NBUF),
    )(_body)
    out = f(idx_blocks, token_table, pos_table)
    return out.transpose(1, 0, 2)


def kernel(input_ids, token_table, pos_table):
    idx_blocks = input_ids.astype(jnp.int32).reshape(
        NUM_WORKERS, BLOCK, SEQ).transpose(0, 2, 1)
    return _run(idx_blocks, token_table, pos_table)


# speedup vs baseline: 1.0078x; 1.0078x over previous
import functools

import jax
import jax.numpy as jnp
from jax import lax
from jax.experimental import pallas as pl
from jax.experimental.pallas import tpu as pltpu
from jax.experimental.pallas import tpu_sc as plsc

VOCAB = 49408
EMBED = 256
BATCH = 4096
SEQ = 77

NUM_CORES = 2
NUM_SUBCORES = 16
NUM_WORKERS = NUM_CORES * NUM_SUBCORES
BLOCK = BATCH // NUM_WORKERS
NBUF = 3
LANES = 16


def _body(idx_hbm, token_hbm, pos_hbm, out_hbm, idx_v, pos_v,
          b0, b1, b2, g0, g1, g2, s0, s1, s2):
    bufs = (b0, b1, b2)
    gsems = (g0, g1, g2)
    ssems = (s0, s1, s2)

    wid = lax.axis_index("s") * NUM_CORES + lax.axis_index("c")
    base = wid * BLOCK

    pltpu.sync_copy(idx_hbm.at[wid], idx_v)

    def gather_start(s, b):
        pltpu.async_copy(token_hbm.at[idx_v.at[s]], bufs[b], gsems[b])

    def gather_wait(b):
        pltpu.make_async_copy(token_hbm.at[pl.ds(0, BLOCK)], bufs[b], gsems[b]).wait()

    def store_wait(b):
        pltpu.make_async_copy(bufs[b], out_hbm.at[0, pl.ds(base, BLOCK)], ssems[b]).wait()

    def add_pos_store(s, b):
        buf = bufs[b]
        regs = [pos_v[s, pl.ds(c * LANES, LANES)] for c in range(EMBED // LANES)]
        half = BLOCK // 2

        @plsc.parallel_loop(0, half, 1, unroll=4)
        def _(i):
            for c in range(EMBED // LANES):
                plsc.addupdate(buf.at[i, pl.ds(c * LANES, LANES)], regs[c])

        pltpu.async_copy(buf.at[pl.ds(0, half)],
                         out_hbm.at[s, pl.ds(base, half)], ssems[b])

        @plsc.parallel_loop(half, BLOCK, 1, unroll=4)
        def _(i):
            for c in range(EMBED // LANES):
                plsc.addupdate(buf.at[i, pl.ds(c * LANES, LANES)], regs[c])

        pltpu.async_copy(buf.at[pl.ds(half, half)],
                         out_hbm.at[s, pl.ds(base + half, half)], ssems[b])

    for b in range(NBUF - 1):
        gather_start(b, b)
    pltpu.sync_copy(pos_hbm, pos_v)

    for b in range(NBUF):
        s = b
        if s == 0:
            gather_start(NBUF - 1, NBUF - 1)
        else:
            store_wait((b - 1) % NBUF)
            gather_start(s + NBUF - 1, (b - 1) % NBUF)
        gather_wait(b)
        add_pos_store(s, b)

    def group(g, carry):
        s0_ = g * NBUF
        for b in range(NBUF):
            s = s0_ + b
            store_wait((b - 1) % NBUF)
            gather_start(s + NBUF - 1, (b - 1) % NBUF)
            gather_wait(b)
            add_pos_store(s, b)
        return carry

    lax.fori_loop(1, (SEQ - (NBUF - 1) - NBUF) // NBUF + 1, group, 0)

    for s in range(SEQ - ((SEQ - NBUF) % NBUF), SEQ):
        b = s % NBUF
        store_wait((b - 1) % NBUF)
        gather_wait(b)
        add_pos_store(s, b)

    store_wait((SEQ - 1) % NBUF)


@jax.jit
def _run(idx_blocks, token_table, pos_table):
    mesh = plsc.VectorSubcoreMesh(core_axis_name="c", subcore_axis_name="s")
    f = functools.partial(
        pl.kernel,
        out_type=jax.ShapeDtypeStruct((SEQ, BATCH, EMBED), jnp.float32),
        mesh=mesh,
        scratch_types=[
            pltpu.VMEM((SEQ, BLOCK), jnp.int32),
            pltpu.VMEM((SEQ, EMBED), jnp.float32),
        ] + [pltpu.VMEM((BLOCK, EMBED), jnp.float32)] * NBUF
          + [pltpu.SemaphoreType.DMA] * (2 * NBUF),
    )(_body)
    out = f(idx_blocks, token_table, pos_table)
    return out.transpose(1, 0, 2)


def kernel(input_ids, token_table, pos_table):
    idx_blocks = input_ids.astype(jnp.int32).reshape(
        NUM_WORKERS, BLOCK, SEQ).transpose(0, 2, 1)
    return _run(idx_blocks, token_table, pos_table)


# speedup vs baseline: 1.0209x; 1.0129x over previous
import functools

import jax
import jax.numpy as jnp
from jax import lax
from jax.experimental import pallas as pl
from jax.experimental.pallas import tpu as pltpu
from jax.experimental.pallas import tpu_sc as plsc

VOCAB = 49408
EMBED = 256
BATCH = 4096
SEQ = 77

NUM_CORES = 2
NUM_SUBCORES = 16
NUM_WORKERS = NUM_CORES * NUM_SUBCORES
BLOCK = BATCH // NUM_WORKERS
NBUF = 3
LANES = 16


def _body(idx_hbm, token_hbm, pos_hbm, out_hbm, idx_v, pos_v,
          b0, b1, b2, g0, g1, g2, s0, s1, s2):
    bufs = (b0, b1, b2)
    gsems = (g0, g1, g2)
    ssems = (s0, s1, s2)

    wid = lax.axis_index("s") * NUM_CORES + lax.axis_index("c")
    base = wid * BLOCK

    pltpu.sync_copy(idx_hbm.at[wid], idx_v)

    def gather_start(s, b):
        pltpu.async_copy(token_hbm.at[idx_v.at[s]], bufs[b], gsems[b])

    def gather_wait(b):
        pltpu.make_async_copy(token_hbm.at[pl.ds(0, BLOCK)], bufs[b], gsems[b]).wait()

    def store_wait(b):
        pltpu.make_async_copy(bufs[b], out_hbm.at[0, pl.ds(base, BLOCK)], ssems[b]).wait()

    def add_pos_store(s, b):
        buf = bufs[b]
        regs = [pos_v[s, pl.ds(c * LANES, LANES)] for c in range(EMBED // LANES)]
        half = BLOCK // 2

        @plsc.parallel_loop(0, half, 1, unroll=2)
        def _(i):
            for c in range(EMBED // LANES):
                plsc.addupdate(buf.at[i, pl.ds(c * LANES, LANES)], regs[c])

        pltpu.async_copy(buf.at[pl.ds(0, half)],
                         out_hbm.at[s, pl.ds(base, half)], ssems[b])

        @plsc.parallel_loop(half, BLOCK, 1, unroll=2)
        def _(i):
            for c in range(EMBED // LANES):
                plsc.addupdate(buf.at[i, pl.ds(c * LANES, LANES)], regs[c])

        pltpu.async_copy(buf.at[pl.ds(half, half)],
                         out_hbm.at[s, pl.ds(base + half, half)], ssems[b])

    for b in range(NBUF - 1):
        gather_start(b, b)
    pltpu.sync_copy(pos_hbm, pos_v)

    for b in range(NBUF):
        s = b
        if s == 0:
            gather_start(NBUF - 1, NBUF - 1)
        else:
            store_wait((b - 1) % NBUF)
            gather_start(s + NBUF - 1, (b - 1) % NBUF)
        gather_wait(b)
        add_pos_store(s, b)

    def group(g, carry):
        s0_ = g * NBUF
        for b in range(NBUF):
            s = s0_ + b
            store_wait((b - 1) % NBUF)
            gather_start(s + NBUF - 1, (b - 1) % NBUF)
            gather_wait(b)
            add_pos_store(s, b)
        return carry

    lax.fori_loop(1, (SEQ - (NBUF - 1) - NBUF) // NBUF + 1, group, 0)

    for s in range(SEQ - ((SEQ - NBUF) % NBUF), SEQ):
        b = s % NBUF
        store_wait((b - 1) % NBUF)
        gather_wait(b)
        add_pos_store(s, b)

    store_wait((SEQ - 1) % NBUF)


@jax.jit
def _run(idx_blocks, token_table, pos_table):
    mesh = plsc.VectorSubcoreMesh(core_axis_name="c", subcore_axis_name="s")
    f = functools.partial(
        pl.kernel,
        out_type=jax.ShapeDtypeStruct((SEQ, BATCH, EMBED), jnp.float32),
        mesh=mesh,
        scratch_types=[
            pltpu.VMEM((SEQ, BLOCK), jnp.int32),
            pltpu.VMEM((SEQ, EMBED), jnp.float32),
        ] + [pltpu.VMEM((BLOCK, EMBED), jnp.float32)] * NBUF
          + [pltpu.SemaphoreType.DMA] * (2 * NBUF),
    )(_body)
    out = f(idx_blocks, token_table, pos_table)
    return out.transpose(1, 0, 2)


def kernel(input_ids, token_table, pos_table):
    idx_blocks = input_ids.astype(jnp.int32).reshape(
        NUM_WORKERS, BLOCK, SEQ).transpose(0, 2, 1)
    return _run(idx_blocks, token_table, pos_table)
